# baseline (speedup 1.0000x reference)
import functools

import jax
import jax.numpy as jnp
from jax import lax
from jax.experimental import pallas as pl
from jax.experimental.pallas import tpu as pltpu
from jax.experimental.pallas import tpu_sc as plsc

NC = 2
NS = 16
NW = NC * NS
CH = 128


def _leaky(x):
    return jnp.where(x >= 0, x, 0.01 * x)


def _ln(x, g, b):
    m = jnp.mean(x, axis=-1, keepdims=True)
    v = jnp.mean((x - m) ** 2, axis=-1, keepdims=True)
    return (x - m) * jax.lax.rsqrt(v + 1e-5) * g + b


def _sc_gather(x, src, dst):
    N, D = x.shape
    E = src.shape[0]
    dt = x.dtype
    per_w = E // NW
    assert per_w * NW == E and per_w % 8 == 0
    npair = per_w // CH // 2

    mesh = plsc.VectorSubcoreMesh(core_axis_name="c", subcore_axis_name="s")

    @functools.partial(
        pl.kernel,
        out_type=[
            jax.ShapeDtypeStruct((E, D), dt),
            jax.ShapeDtypeStruct((E, D), dt),
        ],
        mesh=mesh,
        scratch_types=[
            pltpu.VMEM((per_w,), jnp.int32),
            pltpu.VMEM((per_w,), jnp.int32),
            pltpu.VMEM((CH, D), dt),
            pltpu.VMEM((CH, D), dt),
            pltpu.VMEM((CH, D), dt),
            pltpu.VMEM((CH, D), dt),
            pltpu.SemaphoreType.DMA,
            pltpu.SemaphoreType.DMA,
            pltpu.SemaphoreType.DMA,
            pltpu.SemaphoreType.DMA,
        ],
    )
    def gk(x_hbm, src_hbm, dst_hbm, xs_hbm, xd_hbm, sia, dia,
           sb0, db0, sb1, db1, semg0, semg1, sems0, sems1):
        c = lax.axis_index("c")
        s = lax.axis_index("s")
        base = (s * NC + c) * per_w
        pltpu.sync_copy(src_hbm.at[pl.ds(pl.multiple_of(base, 8), per_w)], sia)
        pltpu.sync_copy(dst_hbm.at[pl.ds(pl.multiple_of(base, 8), per_w)], dia)

        def gather_pair(o0, o1, st0, st1):
            g0a = pltpu.async_copy(x_hbm.at[sia.at[pl.ds(o0, CH)]], sb0, semg0)
            g0b = pltpu.async_copy(x_hbm.at[dia.at[pl.ds(o0, CH)]], db0, semg0)
            g1a = pltpu.async_copy(x_hbm.at[sia.at[pl.ds(o1, CH)]], sb1, semg1)
            g1b = pltpu.async_copy(x_hbm.at[dia.at[pl.ds(o1, CH)]], db1, semg1)
            g0a.wait()
            g0b.wait()
            s0a = pltpu.async_copy(sb0, xs_hbm.at[pl.ds(st0, CH)], sems0)
            s0b = pltpu.async_copy(db0, xd_hbm.at[pl.ds(st0, CH)], sems0)
            g1a.wait()
            g1b.wait()
            s1a = pltpu.async_copy(sb1, xs_hbm.at[pl.ds(st1, CH)], sems1)
            s1b = pltpu.async_copy(db1, xd_hbm.at[pl.ds(st1, CH)], sems1)
            s0a.wait()
            s0b.wait()
            s1a.wait()
            s1b.wait()

        def step(i, carry):
            o0 = pl.multiple_of((2 * i) * CH, 8)
            o1 = pl.multiple_of((2 * i + 1) * CH, 8)
            gather_pair(o0, o1, pl.multiple_of(base + o0, 8),
                        pl.multiple_of(base + o1, 8))
            return carry

        lax.fori_loop(0, npair, step, 0)
        rem = per_w - npair * 2 * CH
        if rem > 0:
            o0 = pl.multiple_of(per_w - CH, 8)
            st0 = pl.multiple_of(base + per_w - CH, 8)
            g0a = pltpu.async_copy(x_hbm.at[sia.at[pl.ds(o0, CH)]], sb0, semg0)
            g0b = pltpu.async_copy(x_hbm.at[dia.at[pl.ds(o0, CH)]], db0, semg0)
            g0a.wait()
            g0b.wait()
            pltpu.sync_copy(sb0, xs_hbm.at[pl.ds(st0, CH)])
            pltpu.sync_copy(db0, xd_hbm.at[pl.ds(st0, CH)])

    return gk(x, src, dst)


def _sc_scatter(m, dst, zeros_nd, counts_only=False):
    N, D = zeros_nd.shape
    E = dst.shape[0]
    per_w = E // NW
    ntail = per_w % CH
    nfull = per_w // CH
    rows_per_sub = N // NS
    assert rows_per_sub * NS == N and N % 16 == 0

    mesh = plsc.VectorSubcoreMesh(core_axis_name="c", subcore_axis_name="s")

    @functools.partial(
        pl.kernel,
        out_type=[jax.ShapeDtypeStruct((NC, N, D), jnp.float32)],
        mesh=mesh,
        scratch_types=[
            pltpu.VMEM((CH,), jnp.int32),
            pltpu.VMEM((CH,), jnp.int32),
            pltpu.VMEM((CH, D), jnp.float32),
            pltpu.VMEM((CH, D), jnp.float32),
            pltpu.VMEM((16,), jnp.int32),
            pltpu.VMEM((16, D), jnp.float32),
            pltpu.VMEM_SHARED((N, D), jnp.float32),
            pltpu.SemaphoreType.DMA,
            pltpu.SemaphoreType.DMA,
            pltpu.SemaphoreType.DMA,
            pltpu.SemaphoreType.DMA,
        ],
    )
    def sk(m_hbm, dst_hbm, z_hbm, part_hbm, idx0, idx1, buf0, buf1, idxt,
           buft, acc, semm0, semm1, sema0, sema1):
        c = lax.axis_index("c")
        s = lax.axis_index("s")
        wid = s * NC + c
        base = wid * per_w
        stripe = ((rows_per_sub + 7) // 8) * 8 + 8
        assert stripe % CH == 0
        r0 = pl.multiple_of((s * rows_per_sub) // 8 * 8, 8)
        r0 = jnp.minimum(r0, N - stripe)
        r0 = pl.multiple_of(r0, 8)

        pltpu.sync_copy(z_hbm.at[pl.ds(0, CH)], buf0)
        for k in range(stripe // CH):
            rk = pl.multiple_of(r0 + k * CH, 8)
            pltpu.sync_copy(buf0, acc.at[pl.ds(rk, CH)])
        if counts_only:
            pltpu.sync_copy(m_hbm, buf0)
            pltpu.sync_copy(m_hbm.at[pl.ds(0, 16)], buft)
        plsc.subcore_barrier()

        assert nfull % 2 == 0

        def step(i, carry):
            st0 = pl.multiple_of(base + (2 * i) * CH, 8)
            st1 = pl.multiple_of(base + (2 * i + 1) * CH, 8)
            pltpu.sync_copy(dst_hbm.at[pl.ds(st0, CH)], idx0)
            pltpu.sync_copy(dst_hbm.at[pl.ds(st1, CH)], idx1)
            if counts_only:
                a0 = pltpu.async_copy(buf0, acc.at[idx0], sema0, add=True)
                a1 = pltpu.async_copy(buf0, acc.at[idx1], sema1, add=True)
            else:
                m0 = pltpu.async_copy(m_hbm.at[pl.ds(st0, CH)], buf0, semm0)
                m1 = pltpu.async_copy(m_hbm.at[pl.ds(st1, CH)], buf1, semm1)
                m0.wait()
                a0 = pltpu.async_copy(buf0, acc.at[idx0], sema0, add=True)
                m1.wait()
                a1 = pltpu.async_copy(buf1, acc.at[idx1], sema1, add=True)
            a0.wait()
            a1.wait()
            return carry

        lax.fori_loop(0, nfull // 2, step, 0)
        if ntail:
            assert ntail == 16
            st = pl.multiple_of(base + nfull * CH, 8)
            pltpu.sync_copy(dst_hbm.at[pl.ds(st, ntail)], idxt)
            if not counts_only:
                pltpu.sync_copy(m_hbm.at[pl.ds(st, ntail)], buft)
            pltpu.sync_copy(buft, acc.at[idxt], add=True)

        plsc.subcore_barrier()
        for k in range(stripe // CH):
            rk = pl.multiple_of(r0 + k * CH, 8)
            pltpu.sync_copy(acc.at[pl.ds(rk, CH)], buf0)
            pltpu.sync_copy(buf0, part_hbm.at[c, pl.ds(rk, CH)])

    return sk(m, dst, zeros_nd)[0]


def _tc_edge(p, xs, xd, ea):
    E, D = xs.shape
    de = ea.shape[1]
    TE = 512
    grid = E // TE
    assert grid * TE == E

    bf = jnp.bfloat16
    w1 = p["edge"]["l1"]["w"]
    w1s, w1d, w1e = w1[:D].astype(bf), w1[D:2 * D].astype(bf), \
        w1[2 * D:].astype(bf)
    b1 = p["edge"]["l1"]["b"][None, :]
    g1 = p["edge"]["ln"]["g"][None, :]
    gb1 = p["edge"]["ln"]["b"][None, :]
    w2 = p["edge"]["l2"]["w"].astype(bf)
    b2 = p["edge"]["l2"]["b"][None, :]
    wn = p["node1"]["l1"]["w"]
    wnx, wne = wn[:D].astype(bf), wn[D:].astype(bf)
    bn1 = p["node1"]["l1"]["b"][None, :]
    gn = p["node1"]["ln"]["g"][None, :]
    gbn = p["node1"]["ln"]["b"][None, :]
    wn2 = p["node1"]["l2"]["w"].astype(bf)
    bn2 = p["node1"]["l2"]["b"][None, :]

    def body(xs_ref, xd_ref, ea_ref, w1s_r, w1d_r, w1e_r, b1_r, g1_r, gb1_r,
             w2_r, b2_r, wnx_r, wne_r, bn1_r, gn_r, gbn_r, wn2_r, bn2_r,
             ean_ref, m_ref):
        bf16 = jnp.bfloat16
        xs_t = xs_ref[...].astype(bf16)
        h = (jnp.dot(xs_t, w1s_r[...], preferred_element_type=jnp.float32)
             + jnp.dot(xd_ref[...].astype(bf16), w1d_r[...],
                       preferred_element_type=jnp.float32)
             + jnp.dot(ea_ref[...].astype(bf16), w1e_r[...],
                       preferred_element_type=jnp.float32)
             + b1_r[...])
        h = _ln(_leaky(h), g1_r[...], gb1_r[...])
        ean = (jnp.dot(h.astype(bf16), w2_r[...],
                       preferred_element_type=jnp.float32) + b2_r[...])
        ean_ref[...] = ean
        h2 = (jnp.dot(xs_t, wnx_r[...], preferred_element_type=jnp.float32)
              + jnp.dot(ean.astype(bf16), wne_r[...],
                        preferred_element_type=jnp.float32)
              + bn1_r[...])
        h2 = _ln(_leaky(h2), gn_r[...], gbn_r[...])
        m_ref[...] = (jnp.dot(h2.astype(bf16), wn2_r[...],
                              preferred_element_type=jnp.float32) + bn2_r[...])

    def cmap(*shape):
        return pl.BlockSpec(shape, lambda i: tuple(0 for _ in shape))

    espec = pl.BlockSpec((TE, D), lambda i: (i, 0))
    return pl.pallas_call(
        body,
        grid=(grid,),
        in_specs=[
            espec, espec, pl.BlockSpec((TE, de), lambda i: (i, 0)),
            cmap(D, D), cmap(D, D), cmap(de, D), cmap(1, D), cmap(1, D),
            cmap(1, D), cmap(D, D), cmap(1, D), cmap(D, D), cmap(D, D),
            cmap(1, D), cmap(1, D), cmap(1, D), cmap(D, D), cmap(1, D),
        ],
        out_specs=[espec, espec],
        out_shape=[
            jax.ShapeDtypeStruct((E, D), jnp.float32),
            jax.ShapeDtypeStruct((E, D), jnp.float32),
        ],
        compiler_params=pltpu.CompilerParams(
            dimension_semantics=("arbitrary",)),
    )(xs, xd, ea, w1s, w1d, w1e, b1, g1, gb1, w2, b2, wnx, wne, bn1, gn, gbn,
      wn2, bn2)


def _tc_node(p, x, part, cnt):
    N, D = x.shape
    TN = 1000
    grid = N // TN
    assert grid * TN == N

    wn = p["node2"]["l1"]["w"]
    wa, wb = wn[:D], wn[D:]
    b1 = p["node2"]["l1"]["b"][None, :]
    g = p["node2"]["ln"]["g"][None, :]
    gb = p["node2"]["ln"]["b"][None, :]
    w2 = p["node2"]["l2"]["w"]
    b2 = p["node2"]["l2"]["b"][None, :]

    def body(x_ref, part_ref, cnt_ref, wa_r, wb_r, b1_r, g_r, gb_r, w2_r,
             b2_r, out_ref):
        cnt = (cnt_ref[0] + cnt_ref[1])[:, :1]
        agg = (part_ref[0] + part_ref[1]) / jnp.maximum(cnt, 1.0)
        h = (jnp.dot(x_ref[...], wa_r[...], preferred_element_type=jnp.float32)
             + jnp.dot(agg, wb_r[...], preferred_element_type=jnp.float32)
             + b1_r[...])
        h = _ln(_leaky(h), g_r[...], gb_r[...])
        out_ref[...] = (jnp.dot(h, w2_r[...],
                                preferred_element_type=jnp.float32) + b2_r[...])

    def cmap(*shape):
        return pl.BlockSpec(shape, lambda i: tuple(0 for _ in shape))

    return pl.pallas_call(
        body,
        grid=(grid,),
        in_specs=[
            pl.BlockSpec((TN, D), lambda i: (i, 0)),
            pl.BlockSpec((NC, TN, D), lambda i: (0, i, 0)),
            pl.BlockSpec((NC, TN, D), lambda i: (0, i, 0)),
            cmap(D, D), cmap(D, D), cmap(1, D), cmap(1, D), cmap(1, D),
            cmap(D, D), cmap(1, D),
        ],
        out_specs=pl.BlockSpec((TN, D), lambda i: (i, 0)),
        out_shape=jax.ShapeDtypeStruct((N, D), jnp.float32),
        compiler_params=pltpu.CompilerParams(
            dimension_semantics=("arbitrary",)),
    )(x, part, cnt, wa, wb, b1, g, gb, w2, b2)


def kernel(x, edge_index, edge_attr, params):
    src = edge_index[0]
    dst = edge_index[1]
    ea = edge_attr
    zeros_nd = jnp.zeros(x.shape, jnp.float32)
    ones_ch = jnp.ones((CH, x.shape[1]), jnp.float32)
    cnt = _sc_scatter(ones_ch, dst, zeros_nd, counts_only=True)
    for lname in ("layer1", "layer2", "layer3"):
        p = params[lname]
        xs, xd = _sc_gather(x, src, dst)
        ean, m = _tc_edge(p, xs, xd, ea)
        part = _sc_scatter(m, dst, zeros_nd)
        x = _tc_node(p, x, part, cnt)
        ea = ean
    return x


# speedup vs baseline: 1.0062x; 1.0062x over previous
import functools

import jax
import jax.numpy as jnp
from jax import lax
from jax.experimental import pallas as pl
from jax.experimental.pallas import tpu as pltpu
from jax.experimental.pallas import tpu_sc as plsc

NC = 2
NS = 16
NW = NC * NS
CH = 128


def _leaky(x):
    return jnp.where(x >= 0, x, 0.01 * x)


def _ln(x, g, b):
    m = jnp.mean(x, axis=-1, keepdims=True)
    v = jnp.mean((x - m) ** 2, axis=-1, keepdims=True)
    return (x - m) * jax.lax.rsqrt(v + 1e-5) * g + b


def _sc_gather(x, src, dst):
    N, D = x.shape
    E = src.shape[0]
    dt = x.dtype
    per_w = E // NW
    assert per_w * NW == E and per_w % 8 == 0
    npair = per_w // CH // 2

    mesh = plsc.VectorSubcoreMesh(core_axis_name="c", subcore_axis_name="s")

    @functools.partial(
        pl.kernel,
        out_type=[
            jax.ShapeDtypeStruct((E, D), dt),
            jax.ShapeDtypeStruct((E, D), dt),
        ],
        mesh=mesh,
        scratch_types=[
            pltpu.VMEM((per_w,), jnp.int32),
            pltpu.VMEM((per_w,), jnp.int32),
            pltpu.VMEM((CH, D), dt),
            pltpu.VMEM((CH, D), dt),
            pltpu.VMEM((CH, D), dt),
            pltpu.VMEM((CH, D), dt),
            pltpu.SemaphoreType.DMA,
            pltpu.SemaphoreType.DMA,
            pltpu.SemaphoreType.DMA,
            pltpu.SemaphoreType.DMA,
        ],
    )
    def gk(x_hbm, src_hbm, dst_hbm, xs_hbm, xd_hbm, sia, dia,
           sb0, db0, sb1, db1, semg0, semg1, sems0, sems1):
        c = lax.axis_index("c")
        s = lax.axis_index("s")
        base = (s * NC + c) * per_w
        pltpu.sync_copy(src_hbm.at[pl.ds(pl.multiple_of(base, 8), per_w)], sia)
        pltpu.sync_copy(dst_hbm.at[pl.ds(pl.multiple_of(base, 8), per_w)], dia)

        def gather_pair(o0, o1, st0, st1):
            g0a = pltpu.async_copy(x_hbm.at[sia.at[pl.ds(o0, CH)]], sb0, semg0)
            g0b = pltpu.async_copy(x_hbm.at[dia.at[pl.ds(o0, CH)]], db0, semg0)
            g1a = pltpu.async_copy(x_hbm.at[sia.at[pl.ds(o1, CH)]], sb1, semg1)
            g1b = pltpu.async_copy(x_hbm.at[dia.at[pl.ds(o1, CH)]], db1, semg1)
            g0a.wait()
            g0b.wait()
            s0a = pltpu.async_copy(sb0, xs_hbm.at[pl.ds(st0, CH)], sems0)
            s0b = pltpu.async_copy(db0, xd_hbm.at[pl.ds(st0, CH)], sems0)
            g1a.wait()
            g1b.wait()
            s1a = pltpu.async_copy(sb1, xs_hbm.at[pl.ds(st1, CH)], sems1)
            s1b = pltpu.async_copy(db1, xd_hbm.at[pl.ds(st1, CH)], sems1)
            s0a.wait()
            s0b.wait()
            s1a.wait()
            s1b.wait()

        def step(i, carry):
            o0 = pl.multiple_of((2 * i) * CH, 8)
            o1 = pl.multiple_of((2 * i + 1) * CH, 8)
            gather_pair(o0, o1, pl.multiple_of(base + o0, 8),
                        pl.multiple_of(base + o1, 8))
            return carry

        lax.fori_loop(0, npair, step, 0)
        rem = per_w - npair * 2 * CH
        if rem > 0:
            o0 = pl.multiple_of(per_w - CH, 8)
            st0 = pl.multiple_of(base + per_w - CH, 8)
            g0a = pltpu.async_copy(x_hbm.at[sia.at[pl.ds(o0, CH)]], sb0, semg0)
            g0b = pltpu.async_copy(x_hbm.at[dia.at[pl.ds(o0, CH)]], db0, semg0)
            g0a.wait()
            g0b.wait()
            pltpu.sync_copy(sb0, xs_hbm.at[pl.ds(st0, CH)])
            pltpu.sync_copy(db0, xd_hbm.at[pl.ds(st0, CH)])

    return gk(x, src, dst)


def _sc_scatter(m, dst, zeros_nd, counts_only=False):
    N, D = zeros_nd.shape
    E = dst.shape[0]
    per_w = E // NW
    ntail = per_w % CH
    nfull = per_w // CH
    rows_per_sub = N // NS
    assert rows_per_sub * NS == N and N % 16 == 0

    mesh = plsc.VectorSubcoreMesh(core_axis_name="c", subcore_axis_name="s")

    @functools.partial(
        pl.kernel,
        out_type=[jax.ShapeDtypeStruct((NC, N, D), jnp.float32)],
        mesh=mesh,
        scratch_types=[
            pltpu.VMEM((CH,), jnp.int32),
            pltpu.VMEM((CH,), jnp.int32),
            pltpu.VMEM((CH, D), jnp.float32),
            pltpu.VMEM((CH, D), jnp.float32),
            pltpu.VMEM((16,), jnp.int32),
            pltpu.VMEM((16, D), jnp.float32),
            pltpu.VMEM_SHARED((N, D), jnp.float32),
            pltpu.SemaphoreType.DMA,
            pltpu.SemaphoreType.DMA,
            pltpu.SemaphoreType.DMA,
            pltpu.SemaphoreType.DMA,
        ],
    )
    def sk(m_hbm, dst_hbm, z_hbm, part_hbm, idx0, idx1, buf0, buf1, idxt,
           buft, acc, semm0, semm1, sema0, sema1):
        c = lax.axis_index("c")
        s = lax.axis_index("s")
        wid = s * NC + c
        base = wid * per_w
        stripe = ((rows_per_sub + 7) // 8) * 8 + 8
        assert stripe % CH == 0
        r0 = pl.multiple_of((s * rows_per_sub) // 8 * 8, 8)
        r0 = jnp.minimum(r0, N - stripe)
        r0 = pl.multiple_of(r0, 8)

        pltpu.sync_copy(z_hbm.at[pl.ds(0, CH)], buf0)
        for k in range(stripe // CH):
            rk = pl.multiple_of(r0 + k * CH, 8)
            pltpu.sync_copy(buf0, acc.at[pl.ds(rk, CH)])
        if counts_only:
            pltpu.sync_copy(m_hbm, buf0)
            pltpu.sync_copy(m_hbm.at[pl.ds(0, 16)], buft)
        plsc.subcore_barrier()

        assert nfull % 2 == 0

        def step(i, carry):
            st0 = pl.multiple_of(base + (2 * i) * CH, 8)
            st1 = pl.multiple_of(base + (2 * i + 1) * CH, 8)
            pltpu.sync_copy(dst_hbm.at[pl.ds(st0, CH)], idx0)
            pltpu.sync_copy(dst_hbm.at[pl.ds(st1, CH)], idx1)
            if counts_only:
                a0 = pltpu.async_copy(buf0, acc.at[idx0], sema0, add=True)
                a1 = pltpu.async_copy(buf0, acc.at[idx1], sema1, add=True)
            else:
                m0 = pltpu.async_copy(m_hbm.at[pl.ds(st0, CH)], buf0, semm0)
                m1 = pltpu.async_copy(m_hbm.at[pl.ds(st1, CH)], buf1, semm1)
                m0.wait()
                a0 = pltpu.async_copy(buf0, acc.at[idx0], sema0, add=True)
                m1.wait()
                a1 = pltpu.async_copy(buf1, acc.at[idx1], sema1, add=True)
            a0.wait()
            a1.wait()
            return carry

        lax.fori_loop(0, nfull // 2, step, 0)
        if ntail:
            assert ntail == 16
            st = pl.multiple_of(base + nfull * CH, 8)
            pltpu.sync_copy(dst_hbm.at[pl.ds(st, ntail)], idxt)
            if not counts_only:
                pltpu.sync_copy(m_hbm.at[pl.ds(st, ntail)], buft)
            pltpu.sync_copy(buft, acc.at[idxt], add=True)

        plsc.subcore_barrier()
        for k in range(stripe // CH):
            rk = pl.multiple_of(r0 + k * CH, 8)
            pltpu.sync_copy(acc.at[pl.ds(rk, CH)], buf0)
            pltpu.sync_copy(buf0, part_hbm.at[c, pl.ds(rk, CH)])

    return sk(m, dst, zeros_nd)[0]


def _tc_edge(p, xs, xd, ea):
    E, D = xs.shape
    de = ea.shape[1]
    TE = 512
    grid = E // TE
    assert grid * TE == E

    bf = jnp.bfloat16
    w1 = p["edge"]["l1"]["w"]
    w1s, w1d, w1e = w1[:D].astype(bf), w1[D:2 * D].astype(bf), \
        w1[2 * D:].astype(bf)
    b1 = p["edge"]["l1"]["b"][None, :]
    g1 = p["edge"]["ln"]["g"][None, :]
    gb1 = p["edge"]["ln"]["b"][None, :]
    w2 = p["edge"]["l2"]["w"].astype(bf)
    b2 = p["edge"]["l2"]["b"][None, :]
    wn = p["node1"]["l1"]["w"]
    wnx, wne = wn[:D].astype(bf), wn[D:].astype(bf)
    bn1 = p["node1"]["l1"]["b"][None, :]
    gn = p["node1"]["ln"]["g"][None, :]
    gbn = p["node1"]["ln"]["b"][None, :]
    wn2 = p["node1"]["l2"]["w"].astype(bf)
    bn2 = p["node1"]["l2"]["b"][None, :]

    def body(xs_ref, xd_ref, ea_ref, w1s_r, w1d_r, w1e_r, b1_r, g1_r, gb1_r,
             w2_r, b2_r, wnx_r, wne_r, bn1_r, gn_r, gbn_r, wn2_r, bn2_r,
             ean_ref, m_ref):
        bf16 = jnp.bfloat16
        xs_t = xs_ref[...].astype(bf16)
        h = (jnp.dot(xs_t, w1s_r[...], preferred_element_type=jnp.float32)
             + jnp.dot(xd_ref[...].astype(bf16), w1d_r[...],
                       preferred_element_type=jnp.float32)
             + jnp.dot(ea_ref[...].astype(bf16), w1e_r[...],
                       preferred_element_type=jnp.float32)
             + b1_r[...])
        h = _ln(_leaky(h), g1_r[...], gb1_r[...])
        ean = (jnp.dot(h.astype(bf16), w2_r[...],
                       preferred_element_type=jnp.float32) + b2_r[...])
        ean_ref[...] = ean.astype(bf16)
        h2 = (jnp.dot(xs_t, wnx_r[...], preferred_element_type=jnp.float32)
              + jnp.dot(ean.astype(bf16), wne_r[...],
                        preferred_element_type=jnp.float32)
              + bn1_r[...])
        h2 = _ln(_leaky(h2), gn_r[...], gbn_r[...])
        m_ref[...] = (jnp.dot(h2.astype(bf16), wn2_r[...],
                              preferred_element_type=jnp.float32) + bn2_r[...])

    def cmap(*shape):
        return pl.BlockSpec(shape, lambda i: tuple(0 for _ in shape))

    espec = pl.BlockSpec((TE, D), lambda i: (i, 0))
    return pl.pallas_call(
        body,
        grid=(grid,),
        in_specs=[
            espec, espec, pl.BlockSpec((TE, de), lambda i: (i, 0)),
            cmap(D, D), cmap(D, D), cmap(de, D), cmap(1, D), cmap(1, D),
            cmap(1, D), cmap(D, D), cmap(1, D), cmap(D, D), cmap(D, D),
            cmap(1, D), cmap(1, D), cmap(1, D), cmap(D, D), cmap(1, D),
        ],
        out_specs=[espec, espec],
        out_shape=[
            jax.ShapeDtypeStruct((E, D), jnp.bfloat16),
            jax.ShapeDtypeStruct((E, D), jnp.float32),
        ],
        compiler_params=pltpu.CompilerParams(
            dimension_semantics=("arbitrary",)),
    )(xs, xd, ea, w1s, w1d, w1e, b1, g1, gb1, w2, b2, wnx, wne, bn1, gn, gbn,
      wn2, bn2)


def _tc_node(p, x, part, cnt):
    N, D = x.shape
    TN = 1000
    grid = N // TN
    assert grid * TN == N

    wn = p["node2"]["l1"]["w"]
    wa, wb = wn[:D], wn[D:]
    b1 = p["node2"]["l1"]["b"][None, :]
    g = p["node2"]["ln"]["g"][None, :]
    gb = p["node2"]["ln"]["b"][None, :]
    w2 = p["node2"]["l2"]["w"]
    b2 = p["node2"]["l2"]["b"][None, :]

    def body(x_ref, part_ref, cnt_ref, wa_r, wb_r, b1_r, g_r, gb_r, w2_r,
             b2_r, out_ref):
        cnt = (cnt_ref[0] + cnt_ref[1])[:, :1]
        agg = (part_ref[0] + part_ref[1]) / jnp.maximum(cnt, 1.0)
        h = (jnp.dot(x_ref[...], wa_r[...], preferred_element_type=jnp.float32)
             + jnp.dot(agg, wb_r[...], preferred_element_type=jnp.float32)
             + b1_r[...])
        h = _ln(_leaky(h), g_r[...], gb_r[...])
        out_ref[...] = (jnp.dot(h, w2_r[...],
                                preferred_element_type=jnp.float32) + b2_r[...])

    def cmap(*shape):
        return pl.BlockSpec(shape, lambda i: tuple(0 for _ in shape))

    return pl.pallas_call(
        body,
        grid=(grid,),
        in_specs=[
            pl.BlockSpec((TN, D), lambda i: (i, 0)),
            pl.BlockSpec((NC, TN, D), lambda i: (0, i, 0)),
            pl.BlockSpec((NC, TN, D), lambda i: (0, i, 0)),
            cmap(D, D), cmap(D, D), cmap(1, D), cmap(1, D), cmap(1, D),
            cmap(D, D), cmap(1, D),
        ],
        out_specs=pl.BlockSpec((TN, D), lambda i: (i, 0)),
        out_shape=jax.ShapeDtypeStruct((N, D), jnp.float32),
        compiler_params=pltpu.CompilerParams(
            dimension_semantics=("arbitrary",)),
    )(x, part, cnt, wa, wb, b1, g, gb, w2, b2)


def kernel(x, edge_index, edge_attr, params):
    src = edge_index[0]
    dst = edge_index[1]
    ea = edge_attr
    zeros_nd = jnp.zeros(x.shape, jnp.float32)
    ones_ch = jnp.ones((CH, x.shape[1]), jnp.float32)
    cnt = _sc_scatter(ones_ch, dst, zeros_nd, counts_only=True)
    for lname in ("layer1", "layer2", "layer3"):
        p = params[lname]
        xs, xd = _sc_gather(x, src, dst)
        ean, m = _tc_edge(p, xs, xd, ea)
        part = _sc_scatter(m, dst, zeros_nd)
        x = _tc_node(p, x, part, cnt)
        ea = ean
    return x


# speedup vs baseline: 1.0217x; 1.0153x over previous
import functools

import jax
import jax.numpy as jnp
from jax import lax
from jax.experimental import pallas as pl
from jax.experimental.pallas import tpu as pltpu
from jax.experimental.pallas import tpu_sc as plsc

NC = 2
NS = 16
NW = NC * NS
CH = 128


def _leaky(x):
    return jnp.where(x >= 0, x, 0.01 * x)


def _ln(x, g, b):
    m = jnp.mean(x, axis=-1, keepdims=True)
    v = jnp.mean((x - m) ** 2, axis=-1, keepdims=True)
    return (x - m) * jax.lax.rsqrt(v + 1e-5) * g + b


def _sc_gather(x, src, dst):
    N, D = x.shape
    E = src.shape[0]
    dt = x.dtype
    per_w = E // NW
    assert per_w * NW == E and per_w % 8 == 0
    npair = per_w // CH // 2

    mesh = plsc.VectorSubcoreMesh(core_axis_name="c", subcore_axis_name="s")

    @functools.partial(
        pl.kernel,
        out_type=[
            jax.ShapeDtypeStruct((E, D), dt),
            jax.ShapeDtypeStruct((E, D), dt),
        ],
        mesh=mesh,
        scratch_types=[
            pltpu.VMEM((per_w,), jnp.int32),
            pltpu.VMEM((per_w,), jnp.int32),
            pltpu.VMEM((CH, D), dt),
            pltpu.VMEM((CH, D), dt),
            pltpu.VMEM((CH, D), dt),
            pltpu.VMEM((CH, D), dt),
            pltpu.SemaphoreType.DMA,
            pltpu.SemaphoreType.DMA,
            pltpu.SemaphoreType.DMA,
            pltpu.SemaphoreType.DMA,
        ],
    )
    def gk(x_hbm, src_hbm, dst_hbm, xs_hbm, xd_hbm, sia, dia,
           sb0, db0, sb1, db1, semg0, semg1, sems0, sems1):
        c = lax.axis_index("c")
        s = lax.axis_index("s")
        base = (s * NC + c) * per_w
        pltpu.sync_copy(src_hbm.at[pl.ds(pl.multiple_of(base, 8), per_w)], sia)
        pltpu.sync_copy(dst_hbm.at[pl.ds(pl.multiple_of(base, 8), per_w)], dia)

        def gather_pair(o0, o1, st0, st1):
            g0a = pltpu.async_copy(x_hbm.at[sia.at[pl.ds(o0, CH)]], sb0, semg0)
            g0b = pltpu.async_copy(x_hbm.at[dia.at[pl.ds(o0, CH)]], db0, semg0)
            g1a = pltpu.async_copy(x_hbm.at[sia.at[pl.ds(o1, CH)]], sb1, semg1)
            g1b = pltpu.async_copy(x_hbm.at[dia.at[pl.ds(o1, CH)]], db1, semg1)
            g0a.wait()
            g0b.wait()
            s0a = pltpu.async_copy(sb0, xs_hbm.at[pl.ds(st0, CH)], sems0)
            s0b = pltpu.async_copy(db0, xd_hbm.at[pl.ds(st0, CH)], sems0)
            g1a.wait()
            g1b.wait()
            s1a = pltpu.async_copy(sb1, xs_hbm.at[pl.ds(st1, CH)], sems1)
            s1b = pltpu.async_copy(db1, xd_hbm.at[pl.ds(st1, CH)], sems1)
            s0a.wait()
            s0b.wait()
            s1a.wait()
            s1b.wait()

        def step(i, carry):
            o0 = pl.multiple_of((2 * i) * CH, 8)
            o1 = pl.multiple_of((2 * i + 1) * CH, 8)
            gather_pair(o0, o1, pl.multiple_of(base + o0, 8),
                        pl.multiple_of(base + o1, 8))
            return carry

        lax.fori_loop(0, npair, step, 0)
        rem = per_w - npair * 2 * CH
        if rem > 0:
            o0 = pl.multiple_of(per_w - CH, 8)
            st0 = pl.multiple_of(base + per_w - CH, 8)
            g0a = pltpu.async_copy(x_hbm.at[sia.at[pl.ds(o0, CH)]], sb0, semg0)
            g0b = pltpu.async_copy(x_hbm.at[dia.at[pl.ds(o0, CH)]], db0, semg0)
            g0a.wait()
            g0b.wait()
            pltpu.sync_copy(sb0, xs_hbm.at[pl.ds(st0, CH)])
            pltpu.sync_copy(db0, xd_hbm.at[pl.ds(st0, CH)])

    return gk(x, src, dst)


def _sc_scatter(m, dst, zeros_nd, counts_only=False):
    N, D = zeros_nd.shape
    E = dst.shape[0]
    per_w = E // NW
    ntail = per_w % CH
    nfull = per_w // CH
    rows_per_sub = N // NS
    assert rows_per_sub * NS == N and N % 16 == 0

    mesh = plsc.VectorSubcoreMesh(core_axis_name="c", subcore_axis_name="s")

    @functools.partial(
        pl.kernel,
        out_type=[jax.ShapeDtypeStruct((NC, N, D), jnp.float32)],
        mesh=mesh,
        scratch_types=[
            pltpu.VMEM((CH,), jnp.int32),
            pltpu.VMEM((CH,), jnp.int32),
            pltpu.VMEM((CH, D), jnp.float32),
            pltpu.VMEM((CH, D), jnp.float32),
            pltpu.VMEM((16,), jnp.int32),
            pltpu.VMEM((16, D), jnp.float32),
            pltpu.VMEM_SHARED((N, D), jnp.float32),
            pltpu.SemaphoreType.DMA,
            pltpu.SemaphoreType.DMA,
            pltpu.SemaphoreType.DMA,
            pltpu.SemaphoreType.DMA,
        ],
    )
    def sk(m_hbm, dst_hbm, z_hbm, part_hbm, idx0, idx1, buf0, buf1, idxt,
           buft, acc, semm0, semm1, sema0, sema1):
        c = lax.axis_index("c")
        s = lax.axis_index("s")
        wid = s * NC + c
        base = wid * per_w
        stripe = ((rows_per_sub + 7) // 8) * 8 + 8
        assert stripe % CH == 0
        r0 = pl.multiple_of((s * rows_per_sub) // 8 * 8, 8)
        r0 = jnp.minimum(r0, N - stripe)
        r0 = pl.multiple_of(r0, 8)

        pltpu.sync_copy(z_hbm.at[pl.ds(0, CH)], buf0)
        for k in range(stripe // CH):
            rk = pl.multiple_of(r0 + k * CH, 8)
            pltpu.sync_copy(buf0, acc.at[pl.ds(rk, CH)])
        if counts_only:
            pltpu.sync_copy(m_hbm, buf0)
            pltpu.sync_copy(m_hbm.at[pl.ds(0, 16)], buft)
        plsc.subcore_barrier()

        assert nfull % 2 == 0

        def step(i, carry):
            st0 = pl.multiple_of(base + (2 * i) * CH, 8)
            st1 = pl.multiple_of(base + (2 * i + 1) * CH, 8)
            pltpu.sync_copy(dst_hbm.at[pl.ds(st0, CH)], idx0)
            pltpu.sync_copy(dst_hbm.at[pl.ds(st1, CH)], idx1)
            if counts_only:
                a0 = pltpu.async_copy(buf0, acc.at[idx0], sema0, add=True)
                a1 = pltpu.async_copy(buf0, acc.at[idx1], sema1, add=True)
            else:
                m0 = pltpu.async_copy(m_hbm.at[pl.ds(st0, CH)], buf0, semm0)
                m1 = pltpu.async_copy(m_hbm.at[pl.ds(st1, CH)], buf1, semm1)
                m0.wait()
                a0 = pltpu.async_copy(buf0, acc.at[idx0], sema0, add=True)
                m1.wait()
                a1 = pltpu.async_copy(buf1, acc.at[idx1], sema1, add=True)
            a0.wait()
            a1.wait()
            return carry

        lax.fori_loop(0, nfull // 2, step, 0)
        if ntail:
            assert ntail == 16
            st = pl.multiple_of(base + nfull * CH, 8)
            pltpu.sync_copy(dst_hbm.at[pl.ds(st, ntail)], idxt)
            if not counts_only:
                pltpu.sync_copy(m_hbm.at[pl.ds(st, ntail)], buft)
            pltpu.sync_copy(buft, acc.at[idxt], add=True)

        plsc.subcore_barrier()
        for k in range(stripe // CH):
            rk = pl.multiple_of(r0 + k * CH, 8)
            pltpu.sync_copy(acc.at[pl.ds(rk, CH)], buf0)
            pltpu.sync_copy(buf0, part_hbm.at[c, pl.ds(rk, CH)])

    return sk(m, dst, zeros_nd)[0]


def _tc_edge(p, xs, xd, ea):
    E, D = xs.shape
    de = ea.shape[1]
    TE = 512
    grid = E // TE
    assert grid * TE == E

    w1 = p["edge"]["l1"]["w"]
    w1s, w1d, w1e = w1[:D], w1[D:2 * D], w1[2 * D:]
    b1 = p["edge"]["l1"]["b"][None, :]
    g1 = p["edge"]["ln"]["g"][None, :]
    gb1 = p["edge"]["ln"]["b"][None, :]
    w2 = p["edge"]["l2"]["w"]
    b2 = p["edge"]["l2"]["b"][None, :]
    wn = p["node1"]["l1"]["w"]
    wnx, wne = wn[:D], wn[D:]
    bn1 = p["node1"]["l1"]["b"][None, :]
    gn = p["node1"]["ln"]["g"][None, :]
    gbn = p["node1"]["ln"]["b"][None, :]
    wn2 = p["node1"]["l2"]["w"]
    bn2 = p["node1"]["l2"]["b"][None, :]

    def body(xs_ref, xd_ref, ea_ref, w1s_r, w1d_r, w1e_r, b1_r, g1_r, gb1_r,
             w2_r, b2_r, wnx_r, wne_r, bn1_r, gn_r, gbn_r, wn2_r, bn2_r,
             ean_ref, m_ref):
        xs_t = xs_ref[...]
        h = (jnp.dot(xs_t, w1s_r[...], preferred_element_type=jnp.float32)
             + jnp.dot(xd_ref[...], w1d_r[...],
                       preferred_element_type=jnp.float32)
             + jnp.dot(ea_ref[...].astype(jnp.float32), w1e_r[...],
                       preferred_element_type=jnp.float32)
             + b1_r[...])
        h = _ln(_leaky(h), g1_r[...], gb1_r[...])
        ean = (jnp.dot(h, w2_r[...],
                       preferred_element_type=jnp.float32) + b2_r[...])
        ean_ref[...] = ean.astype(jnp.bfloat16)
        h2 = (jnp.dot(xs_t, wnx_r[...], preferred_element_type=jnp.float32)
              + jnp.dot(ean, wne_r[...], preferred_element_type=jnp.float32)
              + bn1_r[...])
        h2 = _ln(_leaky(h2), gn_r[...], gbn_r[...])
        m_ref[...] = (jnp.dot(h2, wn2_r[...],
                              preferred_element_type=jnp.float32) + bn2_r[...])

    def cmap(*shape):
        return pl.BlockSpec(shape, lambda i: tuple(0 for _ in shape))

    espec = pl.BlockSpec((TE, D), lambda i: (i, 0))
    return pl.pallas_call(
        body,
        grid=(grid,),
        in_specs=[
            espec, espec, pl.BlockSpec((TE, de), lambda i: (i, 0)),
            cmap(D, D), cmap(D, D), cmap(de, D), cmap(1, D), cmap(1, D),
            cmap(1, D), cmap(D, D), cmap(1, D), cmap(D, D), cmap(D, D),
            cmap(1, D), cmap(1, D), cmap(1, D), cmap(D, D), cmap(1, D),
        ],
        out_specs=[espec, espec],
        out_shape=[
            jax.ShapeDtypeStruct((E, D), jnp.bfloat16),
            jax.ShapeDtypeStruct((E, D), jnp.float32),
        ],
        compiler_params=pltpu.CompilerParams(
            dimension_semantics=("arbitrary",)),
    )(xs, xd, ea, w1s, w1d, w1e, b1, g1, gb1, w2, b2, wnx, wne, bn1, gn, gbn,
      wn2, bn2)


def _tc_node(p, x, part, cnt):
    N, D = x.shape
    TN = 1000
    grid = N // TN
    assert grid * TN == N

    wn = p["node2"]["l1"]["w"]
    wa, wb = wn[:D], wn[D:]
    b1 = p["node2"]["l1"]["b"][None, :]
    g = p["node2"]["ln"]["g"][None, :]
    gb = p["node2"]["ln"]["b"][None, :]
    w2 = p["node2"]["l2"]["w"]
    b2 = p["node2"]["l2"]["b"][None, :]

    def body(x_ref, part_ref, cnt_ref, wa_r, wb_r, b1_r, g_r, gb_r, w2_r,
             b2_r, out_ref):
        cnt = (cnt_ref[0] + cnt_ref[1])[:, :1]
        agg = (part_ref[0] + part_ref[1]) / jnp.maximum(cnt, 1.0)
        h = (jnp.dot(x_ref[...], wa_r[...], preferred_element_type=jnp.float32)
             + jnp.dot(agg, wb_r[...], preferred_element_type=jnp.float32)
             + b1_r[...])
        h = _ln(_leaky(h), g_r[...], gb_r[...])
        out_ref[...] = (jnp.dot(h, w2_r[...],
                                preferred_element_type=jnp.float32) + b2_r[...])

    def cmap(*shape):
        return pl.BlockSpec(shape, lambda i: tuple(0 for _ in shape))

    return pl.pallas_call(
        body,
        grid=(grid,),
        in_specs=[
            pl.BlockSpec((TN, D), lambda i: (i, 0)),
            pl.BlockSpec((NC, TN, D), lambda i: (0, i, 0)),
            pl.BlockSpec((NC, TN, D), lambda i: (0, i, 0)),
            cmap(D, D), cmap(D, D), cmap(1, D), cmap(1, D), cmap(1, D),
            cmap(D, D), cmap(1, D),
        ],
        out_specs=pl.BlockSpec((TN, D), lambda i: (i, 0)),
        out_shape=jax.ShapeDtypeStruct((N, D), jnp.float32),
        compiler_params=pltpu.CompilerParams(
            dimension_semantics=("arbitrary",)),
    )(x, part, cnt, wa, wb, b1, g, gb, w2, b2)


def kernel(x, edge_index, edge_attr, params):
    src = edge_index[0]
    dst = edge_index[1]
    ea = edge_attr
    zeros_nd = jnp.zeros(x.shape, jnp.float32)
    ones_ch = jnp.ones((CH, x.shape[1]), jnp.float32)
    cnt = _sc_scatter(ones_ch, dst, zeros_nd, counts_only=True)
    for lname in ("layer1", "layer2", "layer3"):
        p = params[lname]
        xs, xd = _sc_gather(x, src, dst)
        ean, m = _tc_edge(p, xs, xd, ea)
        part = _sc_scatter(m, dst, zeros_nd)
        x = _tc_node(p, x, part, cnt)
        ea = ean
    return x


# speedup vs baseline: 1.4511x; 1.4203x over previous
import functools

import jax
import jax.numpy as jnp
from jax import lax
from jax.experimental import pallas as pl
from jax.experimental.pallas import tpu as pltpu
from jax.experimental.pallas import tpu_sc as plsc

NC = 2
NS = 16
NW = NC * NS
CH = 128


def _leaky(x):
    return jnp.where(x >= 0, x, 0.01 * x)


def _ln(x, g, b):
    m = jnp.mean(x, axis=-1, keepdims=True)
    v = jnp.mean((x - m) ** 2, axis=-1, keepdims=True)
    return (x - m) * jax.lax.rsqrt(v + 1e-5) * g + b


def _sc_gather(x, src, dst):
    N, D = x.shape
    E = src.shape[0]
    dt = x.dtype
    per_w = E // NW
    assert per_w * NW == E and per_w % 8 == 0
    npair = per_w // CH // 2

    mesh = plsc.VectorSubcoreMesh(core_axis_name="c", subcore_axis_name="s")

    @functools.partial(
        pl.kernel,
        out_type=[
            jax.ShapeDtypeStruct((E, D), dt),
            jax.ShapeDtypeStruct((E, D), dt),
        ],
        mesh=mesh,
        scratch_types=[
            pltpu.VMEM((per_w,), jnp.int32),
            pltpu.VMEM((per_w,), jnp.int32),
            pltpu.VMEM((CH, D), dt),
            pltpu.VMEM((CH, D), dt),
            pltpu.VMEM((CH, D), dt),
            pltpu.VMEM((CH, D), dt),
            pltpu.SemaphoreType.DMA,
            pltpu.SemaphoreType.DMA,
            pltpu.SemaphoreType.DMA,
            pltpu.SemaphoreType.DMA,
        ],
    )
    def gk(x_hbm, src_hbm, dst_hbm, xs_hbm, xd_hbm, sia, dia,
           sb0, db0, sb1, db1, semg0, semg1, sems0, sems1):
        c = lax.axis_index("c")
        s = lax.axis_index("s")
        base = (s * NC + c) * per_w
        pltpu.sync_copy(src_hbm.at[pl.ds(pl.multiple_of(base, 8), per_w)], sia)
        pltpu.sync_copy(dst_hbm.at[pl.ds(pl.multiple_of(base, 8), per_w)], dia)

        def gather_pair(o0, o1, st0, st1):
            g0a = pltpu.async_copy(x_hbm.at[sia.at[pl.ds(o0, CH)]], sb0, semg0)
            g0b = pltpu.async_copy(x_hbm.at[dia.at[pl.ds(o0, CH)]], db0, semg0)
            g1a = pltpu.async_copy(x_hbm.at[sia.at[pl.ds(o1, CH)]], sb1, semg1)
            g1b = pltpu.async_copy(x_hbm.at[dia.at[pl.ds(o1, CH)]], db1, semg1)
            g0a.wait()
            g0b.wait()
            s0a = pltpu.async_copy(sb0, xs_hbm.at[pl.ds(st0, CH)], sems0)
            s0b = pltpu.async_copy(db0, xd_hbm.at[pl.ds(st0, CH)], sems0)
            g1a.wait()
            g1b.wait()
            s1a = pltpu.async_copy(sb1, xs_hbm.at[pl.ds(st1, CH)], sems1)
            s1b = pltpu.async_copy(db1, xd_hbm.at[pl.ds(st1, CH)], sems1)
            s0a.wait()
            s0b.wait()
            s1a.wait()
            s1b.wait()

        def step(i, carry):
            o0 = pl.multiple_of((2 * i) * CH, 8)
            o1 = pl.multiple_of((2 * i + 1) * CH, 8)
            gather_pair(o0, o1, pl.multiple_of(base + o0, 8),
                        pl.multiple_of(base + o1, 8))
            return carry

        lax.fori_loop(0, npair, step, 0)
        rem = per_w - npair * 2 * CH
        if rem > 0:
            o0 = pl.multiple_of(per_w - CH, 8)
            st0 = pl.multiple_of(base + per_w - CH, 8)
            g0a = pltpu.async_copy(x_hbm.at[sia.at[pl.ds(o0, CH)]], sb0, semg0)
            g0b = pltpu.async_copy(x_hbm.at[dia.at[pl.ds(o0, CH)]], db0, semg0)
            g0a.wait()
            g0b.wait()
            pltpu.sync_copy(sb0, xs_hbm.at[pl.ds(st0, CH)])
            pltpu.sync_copy(db0, xd_hbm.at[pl.ds(st0, CH)])

    return gk(x, src, dst)


def _sc_scatter(m, dst, zeros_nd, counts_only=False):
    N, D = zeros_nd.shape
    E = dst.shape[0]
    per_w = E // NW
    ntail = per_w % CH
    nfull = per_w // CH
    rows_per_sub = N // NS
    assert rows_per_sub * NS == N and N % 16 == 0

    mesh = plsc.VectorSubcoreMesh(core_axis_name="c", subcore_axis_name="s")

    @functools.partial(
        pl.kernel,
        out_type=[jax.ShapeDtypeStruct((NC, N, D), jnp.float32)],
        mesh=mesh,
        scratch_types=[
            pltpu.VMEM((CH,), jnp.int32),
            pltpu.VMEM((CH,), jnp.int32),
            pltpu.VMEM((CH, D), jnp.float32),
            pltpu.VMEM((CH, D), jnp.float32),
            pltpu.VMEM((16,), jnp.int32),
            pltpu.VMEM((16, D), jnp.float32),
            pltpu.VMEM_SHARED((N, D), jnp.float32),
            pltpu.SemaphoreType.DMA,
            pltpu.SemaphoreType.DMA,
            pltpu.SemaphoreType.DMA,
            pltpu.SemaphoreType.DMA,
        ],
    )
    def sk(m_hbm, dst_hbm, z_hbm, part_hbm, idx0, idx1, buf0, buf1, idxt,
           buft, acc, semm0, semm1, sema0, sema1):
        c = lax.axis_index("c")
        s = lax.axis_index("s")
        wid = s * NC + c
        base = wid * per_w
        stripe = ((rows_per_sub + 7) // 8) * 8 + 8
        assert stripe % CH == 0
        r0 = pl.multiple_of((s * rows_per_sub) // 8 * 8, 8)
        r0 = jnp.minimum(r0, N - stripe)
        r0 = pl.multiple_of(r0, 8)

        pltpu.sync_copy(z_hbm.at[pl.ds(0, CH)], buf0)
        for k in range(stripe // CH):
            rk = pl.multiple_of(r0 + k * CH, 8)
            pltpu.sync_copy(buf0, acc.at[pl.ds(rk, CH)])
        if counts_only:
            pltpu.sync_copy(m_hbm, buf0)
            pltpu.sync_copy(m_hbm.at[pl.ds(0, 16)], buft)
        plsc.subcore_barrier()

        assert nfull % 2 == 0

        def step(i, carry):
            st0 = pl.multiple_of(base + (2 * i) * CH, 8)
            st1 = pl.multiple_of(base + (2 * i + 1) * CH, 8)
            pltpu.sync_copy(dst_hbm.at[pl.ds(st0, CH)], idx0)
            pltpu.sync_copy(dst_hbm.at[pl.ds(st1, CH)], idx1)
            if counts_only:
                a0 = pltpu.async_copy(buf0, acc.at[idx0], sema0, add=True)
                a1 = pltpu.async_copy(buf0, acc.at[idx1], sema1, add=True)
            else:
                m0 = pltpu.async_copy(m_hbm.at[pl.ds(st0, CH)], buf0, semm0)
                m1 = pltpu.async_copy(m_hbm.at[pl.ds(st1, CH)], buf1, semm1)
                m0.wait()
                a0 = pltpu.async_copy(buf0, acc.at[idx0], sema0, add=True)
                m1.wait()
                a1 = pltpu.async_copy(buf1, acc.at[idx1], sema1, add=True)
            a0.wait()
            a1.wait()
            return carry

        lax.fori_loop(0, nfull // 2, step, 0)
        if ntail:
            assert ntail == 16
            st = pl.multiple_of(base + nfull * CH, 8)
            pltpu.sync_copy(dst_hbm.at[pl.ds(st, ntail)], idxt)
            if not counts_only:
                pltpu.sync_copy(m_hbm.at[pl.ds(st, ntail)], buft)
            pltpu.sync_copy(buft, acc.at[idxt], add=True)

        plsc.subcore_barrier()
        for k in range(stripe // CH):
            rk = pl.multiple_of(r0 + k * CH, 8)
            pltpu.sync_copy(acc.at[pl.ds(rk, CH)], buf0)
            pltpu.sync_copy(buf0, part_hbm.at[c, pl.ds(rk, CH)])

    return sk(m, dst, zeros_nd)[0]


def _tc_edge(p, xs, xd, ea):
    E, D = xs.shape
    de = ea.shape[1]
    TE = 1600
    grid = E // TE
    assert grid * TE == E

    w1 = p["edge"]["l1"]["w"]
    w1s, w1d, w1e = w1[:D], w1[D:2 * D], w1[2 * D:]
    b1 = p["edge"]["l1"]["b"][None, :]
    g1 = p["edge"]["ln"]["g"][None, :]
    gb1 = p["edge"]["ln"]["b"][None, :]
    w2 = p["edge"]["l2"]["w"]
    b2 = p["edge"]["l2"]["b"][None, :]
    wn = p["node1"]["l1"]["w"]
    wnx, wne = wn[:D], wn[D:]
    bn1 = p["node1"]["l1"]["b"][None, :]
    gn = p["node1"]["ln"]["g"][None, :]
    gbn = p["node1"]["ln"]["b"][None, :]
    wn2 = p["node1"]["l2"]["w"]
    bn2 = p["node1"]["l2"]["b"][None, :]

    def body(xs_ref, xd_ref, ea_ref, w1s_r, w1d_r, w1e_r, b1_r, g1_r, gb1_r,
             w2_r, b2_r, wnx_r, wne_r, bn1_r, gn_r, gbn_r, wn2_r, bn2_r,
             ean_ref, m_ref):
        xs_t = xs_ref[...]
        h = (jnp.dot(xs_t, w1s_r[...], preferred_element_type=jnp.float32)
             + jnp.dot(xd_ref[...], w1d_r[...],
                       preferred_element_type=jnp.float32)
             + jnp.dot(ea_ref[...].astype(jnp.float32), w1e_r[...],
                       preferred_element_type=jnp.float32)
             + b1_r[...])
        h = _ln(_leaky(h), g1_r[...], gb1_r[...])
        ean = (jnp.dot(h, w2_r[...],
                       preferred_element_type=jnp.float32) + b2_r[...])
        ean_ref[...] = ean.astype(jnp.bfloat16)
        h2 = (jnp.dot(xs_t, wnx_r[...], preferred_element_type=jnp.float32)
              + jnp.dot(ean, wne_r[...], preferred_element_type=jnp.float32)
              + bn1_r[...])
        h2 = _ln(_leaky(h2), gn_r[...], gbn_r[...])
        m_ref[...] = (jnp.dot(h2, wn2_r[...],
                              preferred_element_type=jnp.float32) + bn2_r[...])

    def cmap(*shape):
        return pl.BlockSpec(shape, lambda i: tuple(0 for _ in shape))

    espec = pl.BlockSpec((TE, D), lambda i: (i, 0))
    return pl.pallas_call(
        body,
        grid=(grid,),
        in_specs=[
            espec, espec, pl.BlockSpec((TE, de), lambda i: (i, 0)),
            cmap(D, D), cmap(D, D), cmap(de, D), cmap(1, D), cmap(1, D),
            cmap(1, D), cmap(D, D), cmap(1, D), cmap(D, D), cmap(D, D),
            cmap(1, D), cmap(1, D), cmap(1, D), cmap(D, D), cmap(1, D),
        ],
        out_specs=[espec, espec],
        out_shape=[
            jax.ShapeDtypeStruct((E, D), jnp.bfloat16),
            jax.ShapeDtypeStruct((E, D), jnp.float32),
        ],
        compiler_params=pltpu.CompilerParams(
            dimension_semantics=("arbitrary",)),
    )(xs, xd, ea, w1s, w1d, w1e, b1, g1, gb1, w2, b2, wnx, wne, bn1, gn, gbn,
      wn2, bn2)


def _tc_node(p, x, part, cnt):
    N, D = x.shape
    TN = 1000
    grid = N // TN
    assert grid * TN == N

    wn = p["node2"]["l1"]["w"]
    wa, wb = wn[:D], wn[D:]
    b1 = p["node2"]["l1"]["b"][None, :]
    g = p["node2"]["ln"]["g"][None, :]
    gb = p["node2"]["ln"]["b"][None, :]
    w2 = p["node2"]["l2"]["w"]
    b2 = p["node2"]["l2"]["b"][None, :]

    def body(x_ref, part_ref, cnt_ref, wa_r, wb_r, b1_r, g_r, gb_r, w2_r,
             b2_r, out_ref):
        cnt = (cnt_ref[0] + cnt_ref[1])[:, :1]
        agg = (part_ref[0] + part_ref[1]) / jnp.maximum(cnt, 1.0)
        h = (jnp.dot(x_ref[...], wa_r[...], preferred_element_type=jnp.float32)
             + jnp.dot(agg, wb_r[...], preferred_element_type=jnp.float32)
             + b1_r[...])
        h = _ln(_leaky(h), g_r[...], gb_r[...])
        out_ref[...] = (jnp.dot(h, w2_r[...],
                                preferred_element_type=jnp.float32) + b2_r[...])

    def cmap(*shape):
        return pl.BlockSpec(shape, lambda i: tuple(0 for _ in shape))

    return pl.pallas_call(
        body,
        grid=(grid,),
        in_specs=[
            pl.BlockSpec((TN, D), lambda i: (i, 0)),
            pl.BlockSpec((NC, TN, D), lambda i: (0, i, 0)),
            pl.BlockSpec((NC, TN, D), lambda i: (0, i, 0)),
            cmap(D, D), cmap(D, D), cmap(1, D), cmap(1, D), cmap(1, D),
            cmap(D, D), cmap(1, D),
        ],
        out_specs=pl.BlockSpec((TN, D), lambda i: (i, 0)),
        out_shape=jax.ShapeDtypeStruct((N, D), jnp.float32),
        compiler_params=pltpu.CompilerParams(
            dimension_semantics=("arbitrary",)),
    )(x, part, cnt, wa, wb, b1, g, gb, w2, b2)


def kernel(x, edge_index, edge_attr, params):
    src = edge_index[0]
    dst = edge_index[1]
    ea = edge_attr
    zeros_nd = jnp.zeros(x.shape, jnp.float32)
    ones_ch = jnp.ones((CH, x.shape[1]), jnp.float32)
    cnt = _sc_scatter(ones_ch, dst, zeros_nd, counts_only=True)
    for lname in ("layer1", "layer2", "layer3"):
        p = params[lname]
        xs, xd = _sc_gather(x, src, dst)
        ean, m = _tc_edge(p, xs, xd, ea)
        part = _sc_scatter(m, dst, zeros_nd)
        x = _tc_node(p, x, part, cnt)
        ea = ean
    return x


# speedup vs baseline: 1.5588x; 1.0742x over previous
import functools

import jax
import jax.numpy as jnp
from jax import lax
from jax.experimental import pallas as pl
from jax.experimental.pallas import tpu as pltpu
from jax.experimental.pallas import tpu_sc as plsc

NC = 2
NS = 16
NW = NC * NS
CH = 128


def _leaky(x):
    return jnp.where(x >= 0, x, 0.01 * x)


def _ln(x, g, b):
    m = jnp.mean(x, axis=-1, keepdims=True)
    v = jnp.mean((x - m) ** 2, axis=-1, keepdims=True)
    return (x - m) * jax.lax.rsqrt(v + 1e-5) * g + b


def _sc_gather(x, src, dst):
    N, D = x.shape
    E = src.shape[0]
    dt = x.dtype
    per_w = E // NW
    assert per_w * NW == E and per_w % 8 == 0
    npair = per_w // CH // 2

    mesh = plsc.VectorSubcoreMesh(core_axis_name="c", subcore_axis_name="s")

    @functools.partial(
        pl.kernel,
        out_type=[
            jax.ShapeDtypeStruct((E, D), dt),
            jax.ShapeDtypeStruct((E, D), dt),
        ],
        mesh=mesh,
        scratch_types=[
            pltpu.VMEM((per_w,), jnp.int32),
            pltpu.VMEM((per_w,), jnp.int32),
            pltpu.VMEM((CH, D), dt),
            pltpu.VMEM((CH, D), dt),
            pltpu.VMEM((CH, D), dt),
            pltpu.VMEM((CH, D), dt),
            pltpu.SemaphoreType.DMA,
            pltpu.SemaphoreType.DMA,
            pltpu.SemaphoreType.DMA,
            pltpu.SemaphoreType.DMA,
        ],
    )
    def gk(x_hbm, src_hbm, dst_hbm, xs_hbm, xd_hbm, sia, dia,
           sb0, db0, sb1, db1, semg0, semg1, sems0, sems1):
        c = lax.axis_index("c")
        s = lax.axis_index("s")
        base = (s * NC + c) * per_w
        pltpu.sync_copy(src_hbm.at[pl.ds(pl.multiple_of(base, 8), per_w)], sia)
        pltpu.sync_copy(dst_hbm.at[pl.ds(pl.multiple_of(base, 8), per_w)], dia)

        def gather_pair(o0, o1, st0, st1):
            g0a = pltpu.async_copy(x_hbm.at[sia.at[pl.ds(o0, CH)]], sb0, semg0)
            g0b = pltpu.async_copy(x_hbm.at[dia.at[pl.ds(o0, CH)]], db0, semg0)
            g1a = pltpu.async_copy(x_hbm.at[sia.at[pl.ds(o1, CH)]], sb1, semg1)
            g1b = pltpu.async_copy(x_hbm.at[dia.at[pl.ds(o1, CH)]], db1, semg1)
            g0a.wait()
            g0b.wait()
            s0a = pltpu.async_copy(sb0, xs_hbm.at[pl.ds(st0, CH)], sems0)
            s0b = pltpu.async_copy(db0, xd_hbm.at[pl.ds(st0, CH)], sems0)
            g1a.wait()
            g1b.wait()
            s1a = pltpu.async_copy(sb1, xs_hbm.at[pl.ds(st1, CH)], sems1)
            s1b = pltpu.async_copy(db1, xd_hbm.at[pl.ds(st1, CH)], sems1)
            s0a.wait()
            s0b.wait()
            s1a.wait()
            s1b.wait()

        def step(i, carry):
            o0 = pl.multiple_of((2 * i) * CH, 8)
            o1 = pl.multiple_of((2 * i + 1) * CH, 8)
            gather_pair(o0, o1, pl.multiple_of(base + o0, 8),
                        pl.multiple_of(base + o1, 8))
            return carry

        lax.fori_loop(0, npair, step, 0)
        rem = per_w - npair * 2 * CH
        if rem > 0:
            o0 = pl.multiple_of(per_w - CH, 8)
            st0 = pl.multiple_of(base + per_w - CH, 8)
            g0a = pltpu.async_copy(x_hbm.at[sia.at[pl.ds(o0, CH)]], sb0, semg0)
            g0b = pltpu.async_copy(x_hbm.at[dia.at[pl.ds(o0, CH)]], db0, semg0)
            g0a.wait()
            g0b.wait()
            pltpu.sync_copy(sb0, xs_hbm.at[pl.ds(st0, CH)])
            pltpu.sync_copy(db0, xd_hbm.at[pl.ds(st0, CH)])

    return gk(x, src, dst)


def _sc_scatter(m, dst, zeros_nd, counts_only=False):
    N, D = zeros_nd.shape
    E = dst.shape[0]
    per_w = E // NW
    ntail = per_w % CH
    nfull = per_w // CH
    rows_per_sub = N // NS
    assert rows_per_sub * NS == N and N % 16 == 0

    mesh = plsc.VectorSubcoreMesh(core_axis_name="c", subcore_axis_name="s")

    @functools.partial(
        pl.kernel,
        out_type=[jax.ShapeDtypeStruct((NC, N, D), jnp.float32)],
        mesh=mesh,
        scratch_types=[
            pltpu.VMEM((CH,), jnp.int32),
            pltpu.VMEM((CH,), jnp.int32),
            pltpu.VMEM((CH, D), jnp.float32),
            pltpu.VMEM((CH, D), jnp.float32),
            pltpu.VMEM((16,), jnp.int32),
            pltpu.VMEM((16, D), jnp.float32),
            pltpu.VMEM_SHARED((N, D), jnp.float32),
            pltpu.SemaphoreType.DMA,
            pltpu.SemaphoreType.DMA,
            pltpu.SemaphoreType.DMA,
            pltpu.SemaphoreType.DMA,
        ],
    )
    def sk(m_hbm, dst_hbm, z_hbm, part_hbm, idx0, idx1, buf0, buf1, idxt,
           buft, acc, semm0, semm1, sema0, sema1):
        c = lax.axis_index("c")
        s = lax.axis_index("s")
        wid = s * NC + c
        base = wid * per_w
        stripe = ((rows_per_sub + 7) // 8) * 8 + 8
        assert stripe % CH == 0
        r0 = pl.multiple_of((s * rows_per_sub) // 8 * 8, 8)
        r0 = jnp.minimum(r0, N - stripe)
        r0 = pl.multiple_of(r0, 8)

        pltpu.sync_copy(z_hbm.at[pl.ds(0, CH)], buf0)
        for k in range(stripe // CH):
            rk = pl.multiple_of(r0 + k * CH, 8)
            pltpu.sync_copy(buf0, acc.at[pl.ds(rk, CH)])
        if counts_only:
            pltpu.sync_copy(m_hbm, buf0)
            pltpu.sync_copy(m_hbm.at[pl.ds(0, 16)], buft)
        plsc.subcore_barrier()

        assert nfull % 2 == 0

        def step(i, carry):
            st0 = pl.multiple_of(base + (2 * i) * CH, 8)
            st1 = pl.multiple_of(base + (2 * i + 1) * CH, 8)
            pltpu.sync_copy(dst_hbm.at[pl.ds(st0, CH)], idx0)
            pltpu.sync_copy(dst_hbm.at[pl.ds(st1, CH)], idx1)
            if counts_only:
                a0 = pltpu.async_copy(buf0, acc.at[idx0], sema0, add=True)
                a1 = pltpu.async_copy(buf0, acc.at[idx1], sema1, add=True)
            else:
                m0 = pltpu.async_copy(m_hbm.at[pl.ds(st0, CH)], buf0, semm0)
                m1 = pltpu.async_copy(m_hbm.at[pl.ds(st1, CH)], buf1, semm1)
                m0.wait()
                a0 = pltpu.async_copy(buf0, acc.at[idx0], sema0, add=True)
                m1.wait()
                a1 = pltpu.async_copy(buf1, acc.at[idx1], sema1, add=True)
            a0.wait()
            a1.wait()
            return carry

        lax.fori_loop(0, nfull // 2, step, 0)
        if ntail:
            assert ntail == 16
            st = pl.multiple_of(base + nfull * CH, 8)
            pltpu.sync_copy(dst_hbm.at[pl.ds(st, ntail)], idxt)
            if not counts_only:
                pltpu.sync_copy(m_hbm.at[pl.ds(st, ntail)], buft)
            pltpu.sync_copy(buft, acc.at[idxt], add=True)

        plsc.subcore_barrier()
        for k in range(stripe // CH):
            rk = pl.multiple_of(r0 + k * CH, 8)
            pltpu.sync_copy(acc.at[pl.ds(rk, CH)], buf0)
            pltpu.sync_copy(buf0, part_hbm.at[c, pl.ds(rk, CH)])

    return sk(m, dst, zeros_nd)[0]


def _tc_edge(p, xs, xd, ea):
    E, D = xs.shape
    de = ea.shape[1]
    TE = 3200
    grid = E // TE
    assert grid * TE == E

    w1 = p["edge"]["l1"]["w"]
    w1s, w1d, w1e = w1[:D], w1[D:2 * D], w1[2 * D:]
    b1 = p["edge"]["l1"]["b"][None, :]
    g1 = p["edge"]["ln"]["g"][None, :]
    gb1 = p["edge"]["ln"]["b"][None, :]
    w2 = p["edge"]["l2"]["w"]
    b2 = p["edge"]["l2"]["b"][None, :]
    wn = p["node1"]["l1"]["w"]
    wnx, wne = wn[:D], wn[D:]
    bn1 = p["node1"]["l1"]["b"][None, :]
    gn = p["node1"]["ln"]["g"][None, :]
    gbn = p["node1"]["ln"]["b"][None, :]
    wn2 = p["node1"]["l2"]["w"]
    bn2 = p["node1"]["l2"]["b"][None, :]

    def body(xs_ref, xd_ref, ea_ref, w1s_r, w1d_r, w1e_r, b1_r, g1_r, gb1_r,
             w2_r, b2_r, wnx_r, wne_r, bn1_r, gn_r, gbn_r, wn2_r, bn2_r,
             ean_ref, m_ref):
        xs_t = xs_ref[...]
        h = (jnp.dot(xs_t, w1s_r[...], preferred_element_type=jnp.float32)
             + jnp.dot(xd_ref[...], w1d_r[...],
                       preferred_element_type=jnp.float32)
             + jnp.dot(ea_ref[...].astype(jnp.float32), w1e_r[...],
                       preferred_element_type=jnp.float32)
             + b1_r[...])
        h = _ln(_leaky(h), g1_r[...], gb1_r[...])
        ean = (jnp.dot(h, w2_r[...],
                       preferred_element_type=jnp.float32) + b2_r[...])
        ean_ref[...] = ean.astype(jnp.bfloat16)
        h2 = (jnp.dot(xs_t, wnx_r[...], preferred_element_type=jnp.float32)
              + jnp.dot(ean, wne_r[...], preferred_element_type=jnp.float32)
              + bn1_r[...])
        h2 = _ln(_leaky(h2), gn_r[...], gbn_r[...])
        m_ref[...] = (jnp.dot(h2, wn2_r[...],
                              preferred_element_type=jnp.float32) + bn2_r[...])

    def cmap(*shape):
        return pl.BlockSpec(shape, lambda i: tuple(0 for _ in shape))

    espec = pl.BlockSpec((TE, D), lambda i: (i, 0))
    return pl.pallas_call(
        body,
        grid=(grid,),
        in_specs=[
            espec, espec, pl.BlockSpec((TE, de), lambda i: (i, 0)),
            cmap(D, D), cmap(D, D), cmap(de, D), cmap(1, D), cmap(1, D),
            cmap(1, D), cmap(D, D), cmap(1, D), cmap(D, D), cmap(D, D),
            cmap(1, D), cmap(1, D), cmap(1, D), cmap(D, D), cmap(1, D),
        ],
        out_specs=[espec, espec],
        out_shape=[
            jax.ShapeDtypeStruct((E, D), jnp.bfloat16),
            jax.ShapeDtypeStruct((E, D), jnp.float32),
        ],
        compiler_params=pltpu.CompilerParams(
            dimension_semantics=("arbitrary",)),
    )(xs, xd, ea, w1s, w1d, w1e, b1, g1, gb1, w2, b2, wnx, wne, bn1, gn, gbn,
      wn2, bn2)


def _tc_node(p, x, part, cnt):
    N, D = x.shape
    TN = 1000
    grid = N // TN
    assert grid * TN == N

    wn = p["node2"]["l1"]["w"]
    wa, wb = wn[:D], wn[D:]
    b1 = p["node2"]["l1"]["b"][None, :]
    g = p["node2"]["ln"]["g"][None, :]
    gb = p["node2"]["ln"]["b"][None, :]
    w2 = p["node2"]["l2"]["w"]
    b2 = p["node2"]["l2"]["b"][None, :]

    def body(x_ref, part_ref, cnt_ref, wa_r, wb_r, b1_r, g_r, gb_r, w2_r,
             b2_r, out_ref):
        cnt = (cnt_ref[0] + cnt_ref[1])[:, :1]
        agg = (part_ref[0] + part_ref[1]) / jnp.maximum(cnt, 1.0)
        h = (jnp.dot(x_ref[...], wa_r[...], preferred_element_type=jnp.float32)
             + jnp.dot(agg, wb_r[...], preferred_element_type=jnp.float32)
             + b1_r[...])
        h = _ln(_leaky(h), g_r[...], gb_r[...])
        out_ref[...] = (jnp.dot(h, w2_r[...],
                                preferred_element_type=jnp.float32) + b2_r[...])

    def cmap(*shape):
        return pl.BlockSpec(shape, lambda i: tuple(0 for _ in shape))

    return pl.pallas_call(
        body,
        grid=(grid,),
        in_specs=[
            pl.BlockSpec((TN, D), lambda i: (i, 0)),
            pl.BlockSpec((NC, TN, D), lambda i: (0, i, 0)),
            pl.BlockSpec((NC, TN, D), lambda i: (0, i, 0)),
            cmap(D, D), cmap(D, D), cmap(1, D), cmap(1, D), cmap(1, D),
            cmap(D, D), cmap(1, D),
        ],
        out_specs=pl.BlockSpec((TN, D), lambda i: (i, 0)),
        out_shape=jax.ShapeDtypeStruct((N, D), jnp.float32),
        compiler_params=pltpu.CompilerParams(
            dimension_semantics=("arbitrary",)),
    )(x, part, cnt, wa, wb, b1, g, gb, w2, b2)


def kernel(x, edge_index, edge_attr, params):
    src = edge_index[0]
    dst = edge_index[1]
    ea = edge_attr
    zeros_nd = jnp.zeros(x.shape, jnp.float32)
    ones_ch = jnp.ones((CH, x.shape[1]), jnp.float32)
    cnt = _sc_scatter(ones_ch, dst, zeros_nd, counts_only=True)
    for lname in ("layer1", "layer2", "layer3"):
        p = params[lname]
        xs, xd = _sc_gather(x, src, dst)
        ean, m = _tc_edge(p, xs, xd, ea)
        part = _sc_scatter(m, dst, zeros_nd)
        x = _tc_node(p, x, part, cnt)
        ea = ean
    return x
